# baseline (speedup 1.0000x reference)
import jax
import jax.numpy as jnp
from jax import lax
from jax.experimental import pallas as pl
from jax.experimental.pallas import tpu as pltpu
from jax.experimental.pallas import tpu_sc as plsc

MAXLEN = 200
EMB = 32
BATCH = 4096
PADDED_LEN = 2 * MAXLEN + 1

CHUNK_ROWS = 16
CHUNK_OUT = CHUNK_ROWS * MAXLEN
GATHER = 128
NUM_GATHERS = CHUNK_OUT // GATHER
LANES = 16


def _body(pos_hbm, len_hbm, table_hbm, out_hbm, pos_v, len_v, idx_v, rows_v, sem):
    info = plsc.get_sparse_core_info()
    nc = info.num_cores
    nw = nc * info.num_subcores
    rows_per_worker = BATCH // nw
    num_chunks = rows_per_worker // CHUNK_ROWS

    wid = lax.axis_index("s") * nc + lax.axis_index("c")
    base = wid * rows_per_worker
    pltpu.sync_copy(pos_hbm.at[pl.ds(base, rows_per_worker)], pos_v)
    pltpu.sync_copy(len_hbm.at[pl.ds(base, rows_per_worker)], len_v)
    lane = lax.iota(jnp.int32, LANES)

    def chunk_body(c, carry):
        pos_vec = pos_v[pl.ds(c * CHUNK_ROWS, CHUNK_ROWS)]
        len_vec = len_v[pl.ds(c * CHUNK_ROWS, CHUNK_ROWS)]
        start_vec = (MAXLEN + 1) - pos_vec
        for r in range(CHUNK_ROWS):
            start_s = start_vec[r]
            len_s = len_vec[r]
            for g in range(13):
                off = 16 * g if g < 12 else MAXLEN - LANES
                j_vec = lane + off
                rel = jnp.clip(j_vec + start_s, 1, PADDED_LEN)
                idx = jnp.where(j_vec < len_s, rel, 0)
                idx_v[pl.ds(r * MAXLEN + off, LANES)] = idx
        copies = []
        for q in range(NUM_GATHERS):
            copies.append(
                pltpu.async_copy(
                    table_hbm.at[idx_v.at[pl.ds(q * GATHER, GATHER)]],
                    rows_v.at[pl.ds(q * GATHER, GATHER)],
                    sem,
                )
            )
        for cp in copies:
            cp.wait()
        out_base = base * MAXLEN + c * CHUNK_OUT
        pltpu.sync_copy(rows_v, out_hbm.at[pl.ds(out_base, CHUNK_OUT)])
        return carry

    lax.fori_loop(0, num_chunks, chunk_body, 0)


def kernel(positions, lengths, table):
    info = plsc.get_sparse_core_info()
    nw = info.num_cores * info.num_subcores
    rows_per_worker = BATCH // nw
    mesh = plsc.VectorSubcoreMesh(core_axis_name="c", subcore_axis_name="s")
    k = pl.kernel(
        _body,
        out_type=jax.ShapeDtypeStruct((BATCH * MAXLEN, EMB), jnp.float32),
        mesh=mesh,
        compiler_params=pltpu.CompilerParams(use_tc_tiling_on_sc=False),
        scratch_types=[
            pltpu.VMEM((rows_per_worker,), jnp.int32),
            pltpu.VMEM((rows_per_worker,), jnp.int32),
            pltpu.VMEM((CHUNK_OUT,), jnp.int32),
            pltpu.VMEM((CHUNK_OUT, EMB), jnp.float32),
            pltpu.SemaphoreType.DMA,
        ],
    )
    flat = k(positions.astype(jnp.int32), lengths.astype(jnp.int32), table)
    return flat.reshape(BATCH, MAXLEN, EMB)


# speedup vs baseline: 7.5516x; 7.5516x over previous
import jax
import jax.numpy as jnp
from jax import lax
from jax.experimental import pallas as pl
from jax.experimental.pallas import tpu as pltpu
from jax.experimental.pallas import tpu_sc as plsc

MAXLEN = 200
EMB = 32
BATCH = 4096
VOCAB = 2 * MAXLEN + 2
PAD_MAX = 2 * MAXLEN + 1

CHUNK_ROWS = 16
CHUNK_OUT = CHUNK_ROWS * MAXLEN
GATHER = 128
NUM_GATHERS = CHUNK_OUT // GATHER
LANES = 16


def _body(pos_hbm, len_hbm, table_hbm, out_hbm, tab_s, pos_v, len_v, idx_v,
          rows_v, sem):
    info = plsc.get_sparse_core_info()
    nc = info.num_cores
    nw = nc * info.num_subcores
    rows_per_worker = BATCH // nw
    num_chunks = rows_per_worker // CHUNK_ROWS

    sid = lax.axis_index("s")
    wid = sid * nc + lax.axis_index("c")
    base = wid * rows_per_worker

    @pl.when(sid == 0)
    def _():
        pltpu.sync_copy(table_hbm, tab_s)

    pltpu.sync_copy(pos_hbm.at[pl.ds(base, rows_per_worker)], pos_v)
    pltpu.sync_copy(len_hbm.at[pl.ds(base, rows_per_worker)], len_v)
    plsc.subcore_barrier()
    lane = lax.iota(jnp.int32, LANES)

    def chunk_body(c, carry):
        pos_vec = pos_v[pl.ds(c * CHUNK_ROWS, CHUNK_ROWS)]
        len_vec = len_v[pl.ds(c * CHUNK_ROWS, CHUNK_ROWS)]
        start_vec = (MAXLEN + 1) - pos_vec
        for r in range(CHUNK_ROWS):
            start_s = start_vec[r]
            len_s = len_vec[r]
            for g in range(13):
                off = 16 * g if g < 12 else MAXLEN - LANES
                j_vec = lane + off
                rel = jnp.clip(j_vec + start_s, 1, PAD_MAX)
                idx = jnp.where(j_vec < len_s, rel, 0)
                idx_v[pl.ds(r * MAXLEN + off, LANES)] = idx
        copies = []
        for q in range(NUM_GATHERS):
            copies.append(
                pltpu.async_copy(
                    tab_s.at[idx_v.at[pl.ds(q * GATHER, GATHER)]],
                    rows_v.at[pl.ds(q * GATHER, GATHER)],
                    sem,
                )
            )
        for cp in copies:
            cp.wait()
        for r in range(CHUNK_ROWS):
            pltpu.sync_copy(
                rows_v.at[pl.ds(r * MAXLEN, MAXLEN)],
                out_hbm.at[base + c * CHUNK_ROWS + r],
            )
        return carry

    lax.fori_loop(0, num_chunks, chunk_body, 0)


def kernel(positions, lengths, table):
    info = plsc.get_sparse_core_info()
    nw = info.num_cores * info.num_subcores
    rows_per_worker = BATCH // nw
    mesh = plsc.VectorSubcoreMesh(core_axis_name="c", subcore_axis_name="s")
    k = pl.kernel(
        _body,
        out_type=jax.ShapeDtypeStruct((BATCH, MAXLEN, EMB), jnp.float32),
        mesh=mesh,
        compiler_params=pltpu.CompilerParams(use_tc_tiling_on_sc=False),
        scratch_types=[
            pltpu.VMEM_SHARED((VOCAB, EMB), jnp.float32),
            pltpu.VMEM((rows_per_worker,), jnp.int32),
            pltpu.VMEM((rows_per_worker,), jnp.int32),
            pltpu.VMEM((CHUNK_OUT,), jnp.int32),
            pltpu.VMEM((CHUNK_OUT, EMB), jnp.float32),
            pltpu.SemaphoreType.DMA,
        ],
    )
    return k(positions.astype(jnp.int32), lengths.astype(jnp.int32), table)


# speedup vs baseline: 13.6273x; 1.8046x over previous
import jax
import jax.numpy as jnp
from jax import lax
from jax.experimental import pallas as pl
from jax.experimental.pallas import tpu as pltpu
from jax.experimental.pallas import tpu_sc as plsc

MAXLEN = 200
EMB = 32
BATCH = 4096
VOCAB = 2 * MAXLEN + 2
PAD_MAX = 2 * MAXLEN + 1

CHUNK_ROWS = 16
CHUNK_OUT = CHUNK_ROWS * MAXLEN
GATHER = 128
NUM_GATHERS = CHUNK_OUT // GATHER
LANES = 16


def _body(pos_hbm, len_hbm, table_hbm, out_hbm, tab_s, pos_v, len_v, idx_v,
          rows_v, sem):
    info = plsc.get_sparse_core_info()
    nc = info.num_cores
    nw = nc * info.num_subcores
    rows_per_worker = BATCH // nw
    num_chunks = rows_per_worker // CHUNK_ROWS

    sid = lax.axis_index("s")
    wid = sid * nc + lax.axis_index("c")
    base = wid * rows_per_worker

    @pl.when(sid == 0)
    def _():
        pltpu.sync_copy(table_hbm, tab_s)

    pltpu.sync_copy(pos_hbm.at[pl.ds(base, rows_per_worker)], pos_v)
    pltpu.sync_copy(len_hbm.at[pl.ds(base, rows_per_worker)], len_v)
    plsc.subcore_barrier()
    lane = lax.iota(jnp.int32, LANES)

    def chunk_body(c, carry):
        pos_vec = pos_v[pl.ds(c * CHUNK_ROWS, CHUNK_ROWS)]
        len_vec = len_v[pl.ds(c * CHUNK_ROWS, CHUNK_ROWS)]
        start_vec = (MAXLEN + 1) - pos_vec
        for r in range(CHUNK_ROWS):
            start_s = start_vec[r]
            len_s = len_vec[r]
            for g in range(13):
                off = 16 * g if g < 12 else MAXLEN - LANES
                j_vec = lane + off
                rel = jnp.clip(j_vec + start_s, 1, PAD_MAX)
                idx = jnp.where(j_vec < len_s, rel, 0)
                idx_v[pl.ds(r * MAXLEN + off, LANES)] = idx
        copies = []
        for q in range(NUM_GATHERS):
            copies.append(
                pltpu.async_copy(
                    tab_s.at[idx_v.at[pl.ds(q * GATHER, GATHER)]],
                    rows_v.at[pl.ds(q * GATHER, GATHER)],
                    sem,
                )
            )
        for cp in copies:
            cp.wait()
        for r in range(CHUNK_ROWS):
            pltpu.sync_copy(
                rows_v.at[pl.ds(r * MAXLEN, MAXLEN)],
                out_hbm.at[base + c * CHUNK_ROWS + r, :, pl.ds(0, EMB)],
            )
        return carry

    lax.fori_loop(0, num_chunks, chunk_body, 0)


def kernel(positions, lengths, table):
    info = plsc.get_sparse_core_info()
    nw = info.num_cores * info.num_subcores
    rows_per_worker = BATCH // nw
    mesh = plsc.VectorSubcoreMesh(core_axis_name="c", subcore_axis_name="s")
    k = pl.kernel(
        _body,
        out_type=jax.ShapeDtypeStruct((BATCH, MAXLEN, 128), jnp.float32),
        mesh=mesh,
        compiler_params=pltpu.CompilerParams(use_tc_tiling_on_sc=False),
        scratch_types=[
            pltpu.VMEM_SHARED((VOCAB, EMB), jnp.float32),
            pltpu.VMEM((rows_per_worker,), jnp.int32),
            pltpu.VMEM((rows_per_worker,), jnp.int32),
            pltpu.VMEM((CHUNK_OUT,), jnp.int32),
            pltpu.VMEM((CHUNK_OUT, EMB), jnp.float32),
            pltpu.SemaphoreType.DMA,
        ],
    )
    padded = k(positions.astype(jnp.int32), lengths.astype(jnp.int32), table)
    return padded[:, :, :EMB]
